# baseline (speedup 1.0000x reference)
import functools

import jax
import jax.numpy as jnp
from jax import lax
from jax.experimental import pallas as pl
from jax.experimental.pallas import tpu as pltpu
from jax.experimental.pallas import tpu_sc as plsc

_N = 10000
_E = 320000
_D_IN = 128
_D_H = 64

_NC = 2
_NS = 16
_NW = _NC * _NS
_CHUNK = 128
_TOT_CH = 2560
_EP = _TOT_CH * _CHUNK
_CH0 = 30
_CH1 = 130
_CHMAX = max(_CH0, _CH1)
_R = 10112
_RPT = _R // _NS
_DEPTH = 4


def _sc_mesh():
    return plsc.VectorSubcoreMesh(core_axis_name="c", subcore_axis_name="s",
                                  num_cores=_NC, num_subcores=_NS)


def _sc_agg_body(with_count, *refs):
    if with_count:
        (tbl, src3, dst3, zrows, zrows8, ones, aggp, cntp,
         agg_sh, cnt_sh, sidx2, didx2, ones_v, csem,
         *bufs) = refs
    else:
        (tbl, src3, dst3, zrows, aggp,
         agg_sh, sidx2, didx2, *bufs) = refs
    rows = bufs[:_DEPTH]
    gsems = bufs[_DEPTH:2 * _DEPTH]
    ssems = bufs[2 * _DEPTH:]
    c = lax.axis_index("c")
    s = lax.axis_index("s")
    wid = c * _NS + s
    rbase = s * _RPT
    count = jnp.where(c == 0, _CH0, _CH1)

    pltpu.sync_copy(zrows, agg_sh.at[pl.ds(rbase, _RPT)])
    if with_count:
        pltpu.sync_copy(zrows8, cnt_sh.at[pl.ds(rbase, _RPT)])
        pltpu.sync_copy(ones, ones_v)

    @pl.when(c == 0)
    def _():
        pltpu.sync_copy(src3.at[wid, pl.ds(0, _CH0)],
                        sidx2.at[pl.ds(0, _CH0)])
        pltpu.sync_copy(dst3.at[wid, pl.ds(0, _CH0)],
                        didx2.at[pl.ds(0, _CH0)])

    @pl.when(c == 1)
    def _():
        pltpu.sync_copy(src3.at[wid, pl.ds(0, _CH1)],
                        sidx2.at[pl.ds(0, _CH1)])
        pltpu.sync_copy(dst3.at[wid, pl.ds(0, _CH1)],
                        didx2.at[pl.ds(0, _CH1)])

    plsc.subcore_barrier()

    for m in range(_DEPTH - 1):
        pltpu.async_copy(tbl.at[sidx2.at[m]], rows[m], gsems[m])

    def step(j, carry):
        for m in range(_DEPTH):
            @pl.when(j % _DEPTH == m)
            def _(m=m):
                pltpu.make_async_copy(tbl.at[sidx2.at[j]], rows[m],
                                      gsems[m]).wait()
                pltpu.async_copy(rows[m], agg_sh.at[didx2.at[j]], ssems[m],
                                 add=True)
                if with_count:
                    pltpu.async_copy(ones_v, cnt_sh.at[didx2.at[j]], csem,
                                     add=True)
                nxt = j + _DEPTH - 1
                b = (m + _DEPTH - 1) % _DEPTH

                @pl.when(nxt < count)
                def _():
                    @pl.when(j >= 1)
                    def _():
                        pltpu.make_async_copy(rows[b],
                                              agg_sh.at[didx2.at[0]],
                                              ssems[b]).wait()

                    pltpu.async_copy(tbl.at[sidx2.at[nxt]], rows[b],
                                     gsems[b])
        return carry

    lax.fori_loop(0, count, step, 0)

    for m in range(_DEPTH):
        pltpu.make_async_copy(rows[m], agg_sh.at[didx2.at[0]],
                              ssems[m]).wait()
    if with_count:
        def drain(j, carry):
            pltpu.make_async_copy(ones_v, cnt_sh.at[didx2.at[0]],
                                  csem).wait()
            return carry

        lax.fori_loop(0, count, drain, 0)
    plsc.subcore_barrier()

    pltpu.sync_copy(agg_sh.at[pl.ds(rbase, _RPT)],
                    aggp.at[c, pl.ds(rbase, _RPT)])
    if with_count:
        pltpu.sync_copy(cnt_sh.at[pl.ds(rbase, _RPT)],
                        cntp.at[c, pl.ds(rbase, _RPT)])


def _sc_aggregate1(p1, src3, dst3, zrows, zrows8, ones):
    fn = pl.kernel(
        functools.partial(_sc_agg_body, True),
        out_type=(
            jax.ShapeDtypeStruct((_NC, _R, _D_H), jnp.float32),
            jax.ShapeDtypeStruct((_NC, _R, 8), jnp.float32),
        ),
        mesh=_sc_mesh(),
        compiler_params=pltpu.CompilerParams(use_tc_tiling_on_sc=False),
        scratch_types=[
            pltpu.VMEM_SHARED((_R, _D_H), jnp.float32),
            pltpu.VMEM_SHARED((_R, 8), jnp.float32),
            pltpu.VMEM((_CHMAX, _CHUNK), jnp.int32),
            pltpu.VMEM((_CHMAX, _CHUNK), jnp.int32),
            pltpu.VMEM((_CHUNK, 8), jnp.float32),
            pltpu.SemaphoreType.DMA,
        ] + [pltpu.VMEM((_CHUNK, _D_H), jnp.float32)] * _DEPTH
          + [pltpu.SemaphoreType.DMA] * (2 * _DEPTH),
    )
    return fn(p1, src3, dst3, zrows, zrows8, ones)


def _sc_aggregate2(p2, src3, dst3, zrows8):
    fn = pl.kernel(
        functools.partial(_sc_agg_body, False),
        out_type=jax.ShapeDtypeStruct((_NC, _R, 8), jnp.float32),
        mesh=_sc_mesh(),
        compiler_params=pltpu.CompilerParams(use_tc_tiling_on_sc=False),
        scratch_types=[
            pltpu.VMEM_SHARED((_R, 8), jnp.float32),
            pltpu.VMEM((_CHMAX, _CHUNK), jnp.int32),
            pltpu.VMEM((_CHMAX, _CHUNK), jnp.int32),
        ] + [pltpu.VMEM((_CHUNK, 8), jnp.float32)] * _DEPTH
          + [pltpu.SemaphoreType.DMA] * (2 * _DEPTH),
    )
    return fn(p2, src3, dst3, zrows8)


def _tc_proj1_body(x_ref, wl_ref, wr_ref, b_ref, p_ref, r_ref):
    x = x_ref[...]
    p_ref[...] = jnp.dot(x, wl_ref[...], preferred_element_type=jnp.float32)
    r_ref[...] = (jnp.dot(x, wr_ref[...], preferred_element_type=jnp.float32)
                  + b_ref[...])


def _tc_mid_body(a0_ref, a1_ref, c0_ref, c1_ref, r1_ref, wl_ref, wr_ref,
                 b_ref, p2_ref, r2_ref):
    cnt = jnp.maximum(c0_ref[...] + c1_ref[...], 1.0)
    mean = (a0_ref[...] + a1_ref[...]) / cnt
    h = jnp.maximum(mean + r1_ref[...], 0.0)
    p2_ref[...] = jnp.dot(h, wl_ref[...], preferred_element_type=jnp.float32)
    r2_ref[...] = (jnp.dot(h, wr_ref[...], preferred_element_type=jnp.float32)
                   + b_ref[...])


def _tc_out_body(a0_ref, a1_ref, c0_ref, c1_ref, r2_ref, o_ref):
    cnt = jnp.maximum(c0_ref[...] + c1_ref[...], 1.0)
    o_ref[...] = (a0_ref[...] + a1_ref[...]) / cnt + r2_ref[...]


def _edge_layout(vals, pad_val, counts=(_CH0, _CH1)):
    chunks = vals.reshape(_TOT_CH, _CHUNK)
    parts = []
    lo = 0
    for cnt in counts:
        blk = chunks[lo:lo + _NS * cnt].reshape(_NS, cnt, _CHUNK)
        if cnt < _CHMAX:
            fill = jnp.full((_NS, _CHMAX - cnt, _CHUNK), pad_val, jnp.int32)
            blk = jnp.concatenate([blk, fill], axis=1)
        parts.append(blk)
        lo += _NS * cnt
    return jnp.concatenate(parts, axis=0)


def kernel(x, edge_index, W1_l, W1_r, b1, W2_l, W2_r, b2):
    src = edge_index[0]
    dst = edge_index[1]
    pad = _EP - _E
    src3 = _edge_layout(
        jnp.concatenate([src, jnp.zeros((pad,), jnp.int32)]), 0)
    dst3 = _edge_layout(
        jnp.concatenate([dst, jnp.full((pad,), _N, jnp.int32)]), _N)
    zrows = jnp.zeros((_RPT, _D_H), jnp.float32)
    zrows8 = jnp.zeros((_RPT, 8), jnp.float32)
    ones = jnp.ones((_CHUNK, 8), jnp.float32)

    p1, r1 = pl.pallas_call(
        _tc_proj1_body,
        out_shape=(
            jax.ShapeDtypeStruct((_N, _D_H), jnp.float32),
            jax.ShapeDtypeStruct((_N, _D_H), jnp.float32),
        ),
    )(x, W1_l.T, W1_r.T, b1[None, :])

    aggp, cntp = _sc_aggregate1(p1, src3, dst3, zrows, zrows8, ones)

    w2l8 = jnp.zeros((_D_H, 8), jnp.float32).at[:, :2].set(W2_l.T)
    w2r8 = jnp.zeros((_D_H, 8), jnp.float32).at[:, :2].set(W2_r.T)
    b2_8 = jnp.zeros((1, 8), jnp.float32).at[0, :2].set(b2)
    p2, r2 = pl.pallas_call(
        _tc_mid_body,
        out_shape=(
            jax.ShapeDtypeStruct((_N, 8), jnp.float32),
            jax.ShapeDtypeStruct((_N, 8), jnp.float32),
        ),
    )(aggp[0, :_N], aggp[1, :_N], cntp[0, :_N, :1], cntp[1, :_N, :1],
      r1, w2l8, w2r8, b2_8)

    agg2p = _sc_aggregate2(p2, src3, dst3, zrows8)

    out8 = pl.pallas_call(
        _tc_out_body,
        out_shape=jax.ShapeDtypeStruct((_N, 8), jnp.float32),
    )(agg2p[0, :_N], agg2p[1, :_N], cntp[0, :_N, :1], cntp[1, :_N, :1], r2)

    return out8[:, :2]


# speedup vs baseline: 1.2309x; 1.2309x over previous
import functools

import jax
import jax.numpy as jnp
from jax import lax
from jax.experimental import pallas as pl
from jax.experimental.pallas import tpu as pltpu
from jax.experimental.pallas import tpu_sc as plsc

_N = 10000
_E = 320000
_D_IN = 128
_D_H = 64

_NC = 2
_NS = 16
_NW = _NC * _NS
_CHUNK = 128
_TOT_CH = 2560
_EP = _TOT_CH * _CHUNK
_CH0 = 130
_CH1 = 30
_CHMAX = max(_CH0, _CH1)
_R = 10112
_RPT = _R // _NS
_DEPTH = 4


def _sc_mesh():
    return plsc.VectorSubcoreMesh(core_axis_name="c", subcore_axis_name="s",
                                  num_cores=_NC, num_subcores=_NS)


def _sc_agg_body(with_count, *refs):
    if with_count:
        (tbl, src3, dst3, zrows, zrows8, ones, aggp, cntp,
         agg_sh, cnt_sh, sidx2, didx2, ones_v, csem,
         *bufs) = refs
    else:
        (tbl, src3, dst3, zrows, aggp,
         agg_sh, sidx2, didx2, *bufs) = refs
    rows = bufs[:_DEPTH]
    gsems = bufs[_DEPTH:2 * _DEPTH]
    ssems = bufs[2 * _DEPTH:]
    c = lax.axis_index("c")
    s = lax.axis_index("s")
    wid = c * _NS + s
    rbase = s * _RPT
    count = jnp.where(c == 0, _CH0, _CH1)

    pltpu.sync_copy(zrows, agg_sh.at[pl.ds(rbase, _RPT)])
    if with_count:
        pltpu.sync_copy(zrows8, cnt_sh.at[pl.ds(rbase, _RPT)])
        pltpu.sync_copy(ones, ones_v)

    @pl.when(c == 0)
    def _():
        pltpu.sync_copy(src3.at[wid, pl.ds(0, _CH0)],
                        sidx2.at[pl.ds(0, _CH0)])
        pltpu.sync_copy(dst3.at[wid, pl.ds(0, _CH0)],
                        didx2.at[pl.ds(0, _CH0)])

    @pl.when(c == 1)
    def _():
        pltpu.sync_copy(src3.at[wid, pl.ds(0, _CH1)],
                        sidx2.at[pl.ds(0, _CH1)])
        pltpu.sync_copy(dst3.at[wid, pl.ds(0, _CH1)],
                        didx2.at[pl.ds(0, _CH1)])

    plsc.subcore_barrier()

    for m in range(_DEPTH - 1):
        pltpu.async_copy(tbl.at[sidx2.at[m]], rows[m], gsems[m])

    def step(j, carry):
        for m in range(_DEPTH):
            @pl.when(j % _DEPTH == m)
            def _(m=m):
                pltpu.make_async_copy(tbl.at[sidx2.at[j]], rows[m],
                                      gsems[m]).wait()
                pltpu.async_copy(rows[m], agg_sh.at[didx2.at[j]], ssems[m],
                                 add=True)
                if with_count:
                    pltpu.async_copy(ones_v, cnt_sh.at[didx2.at[j]], csem,
                                     add=True)
                nxt = j + _DEPTH - 1
                b = (m + _DEPTH - 1) % _DEPTH

                @pl.when(nxt < count)
                def _():
                    @pl.when(j >= 1)
                    def _():
                        pltpu.make_async_copy(rows[b],
                                              agg_sh.at[didx2.at[0]],
                                              ssems[b]).wait()

                    pltpu.async_copy(tbl.at[sidx2.at[nxt]], rows[b],
                                     gsems[b])
        return carry

    lax.fori_loop(0, count, step, 0)

    for m in range(_DEPTH):
        pltpu.make_async_copy(rows[m], agg_sh.at[didx2.at[0]],
                              ssems[m]).wait()
    if with_count:
        def drain(j, carry):
            pltpu.make_async_copy(ones_v, cnt_sh.at[didx2.at[0]],
                                  csem).wait()
            return carry

        lax.fori_loop(0, count, drain, 0)
    plsc.subcore_barrier()

    pltpu.sync_copy(agg_sh.at[pl.ds(rbase, _RPT)],
                    aggp.at[c, pl.ds(rbase, _RPT)])
    if with_count:
        pltpu.sync_copy(cnt_sh.at[pl.ds(rbase, _RPT)],
                        cntp.at[c, pl.ds(rbase, _RPT)])


def _sc_aggregate1(p1, src3, dst3, zrows, zrows8, ones):
    fn = pl.kernel(
        functools.partial(_sc_agg_body, True),
        out_type=(
            jax.ShapeDtypeStruct((_NC, _R, _D_H), jnp.float32),
            jax.ShapeDtypeStruct((_NC, _R, 8), jnp.float32),
        ),
        mesh=_sc_mesh(),
        compiler_params=pltpu.CompilerParams(use_tc_tiling_on_sc=False),
        scratch_types=[
            pltpu.VMEM_SHARED((_R, _D_H), jnp.float32),
            pltpu.VMEM_SHARED((_R, 8), jnp.float32),
            pltpu.VMEM((_CHMAX, _CHUNK), jnp.int32),
            pltpu.VMEM((_CHMAX, _CHUNK), jnp.int32),
            pltpu.VMEM((_CHUNK, 8), jnp.float32),
            pltpu.SemaphoreType.DMA,
        ] + [pltpu.VMEM((_CHUNK, _D_H), jnp.float32)] * _DEPTH
          + [pltpu.SemaphoreType.DMA] * (2 * _DEPTH),
    )
    return fn(p1, src3, dst3, zrows, zrows8, ones)


def _sc_aggregate2(p2, src3, dst3, zrows8):
    fn = pl.kernel(
        functools.partial(_sc_agg_body, False),
        out_type=jax.ShapeDtypeStruct((_NC, _R, 8), jnp.float32),
        mesh=_sc_mesh(),
        compiler_params=pltpu.CompilerParams(use_tc_tiling_on_sc=False),
        scratch_types=[
            pltpu.VMEM_SHARED((_R, 8), jnp.float32),
            pltpu.VMEM((_CHMAX, _CHUNK), jnp.int32),
            pltpu.VMEM((_CHMAX, _CHUNK), jnp.int32),
        ] + [pltpu.VMEM((_CHUNK, 8), jnp.float32)] * _DEPTH
          + [pltpu.SemaphoreType.DMA] * (2 * _DEPTH),
    )
    return fn(p2, src3, dst3, zrows8)


def _tc_proj1_body(x_ref, wl_ref, wr_ref, b_ref, p_ref, r_ref):
    x = x_ref[...]
    p_ref[...] = jnp.dot(x, wl_ref[...], preferred_element_type=jnp.float32)
    r_ref[...] = (jnp.dot(x, wr_ref[...], preferred_element_type=jnp.float32)
                  + b_ref[...])


def _tc_mid_body(a0_ref, a1_ref, c0_ref, c1_ref, r1_ref, wl_ref, wr_ref,
                 b_ref, p2_ref, r2_ref):
    cnt = jnp.maximum(c0_ref[...] + c1_ref[...], 1.0)
    mean = (a0_ref[...] + a1_ref[...]) / cnt
    h = jnp.maximum(mean + r1_ref[...], 0.0)
    p2_ref[...] = jnp.dot(h, wl_ref[...], preferred_element_type=jnp.float32)
    r2_ref[...] = (jnp.dot(h, wr_ref[...], preferred_element_type=jnp.float32)
                   + b_ref[...])


def _tc_out_body(a0_ref, a1_ref, c0_ref, c1_ref, r2_ref, o_ref):
    cnt = jnp.maximum(c0_ref[...] + c1_ref[...], 1.0)
    o_ref[...] = (a0_ref[...] + a1_ref[...]) / cnt + r2_ref[...]


def _edge_layout(vals, pad_val, counts=(_CH0, _CH1)):
    chunks = vals.reshape(_TOT_CH, _CHUNK)
    parts = []
    lo = 0
    for cnt in counts:
        blk = chunks[lo:lo + _NS * cnt].reshape(_NS, cnt, _CHUNK)
        if cnt < _CHMAX:
            fill = jnp.full((_NS, _CHMAX - cnt, _CHUNK), pad_val, jnp.int32)
            blk = jnp.concatenate([blk, fill], axis=1)
        parts.append(blk)
        lo += _NS * cnt
    return jnp.concatenate(parts, axis=0)


def kernel(x, edge_index, W1_l, W1_r, b1, W2_l, W2_r, b2):
    src = edge_index[0]
    dst = edge_index[1]
    pad = _EP - _E
    src3 = _edge_layout(
        jnp.concatenate([src, jnp.zeros((pad,), jnp.int32)]), 0)
    dst3 = _edge_layout(
        jnp.concatenate([dst, jnp.full((pad,), _N, jnp.int32)]), _N)
    zrows = jnp.zeros((_RPT, _D_H), jnp.float32)
    zrows8 = jnp.zeros((_RPT, 8), jnp.float32)
    ones = jnp.ones((_CHUNK, 8), jnp.float32)

    p1, r1 = pl.pallas_call(
        _tc_proj1_body,
        out_shape=(
            jax.ShapeDtypeStruct((_N, _D_H), jnp.float32),
            jax.ShapeDtypeStruct((_N, _D_H), jnp.float32),
        ),
    )(x, W1_l.T, W1_r.T, b1[None, :])

    aggp, cntp = _sc_aggregate1(p1, src3, dst3, zrows, zrows8, ones)

    w2l8 = jnp.zeros((_D_H, 8), jnp.float32).at[:, :2].set(W2_l.T)
    w2r8 = jnp.zeros((_D_H, 8), jnp.float32).at[:, :2].set(W2_r.T)
    b2_8 = jnp.zeros((1, 8), jnp.float32).at[0, :2].set(b2)
    p2, r2 = pl.pallas_call(
        _tc_mid_body,
        out_shape=(
            jax.ShapeDtypeStruct((_N, 8), jnp.float32),
            jax.ShapeDtypeStruct((_N, 8), jnp.float32),
        ),
    )(aggp[0, :_N], aggp[1, :_N], cntp[0, :_N, :1], cntp[1, :_N, :1],
      r1, w2l8, w2r8, b2_8)

    agg2p = _sc_aggregate2(p2, src3, dst3, zrows8)

    out8 = pl.pallas_call(
        _tc_out_body,
        out_shape=jax.ShapeDtypeStruct((_N, 8), jnp.float32),
    )(agg2p[0, :_N], agg2p[1, :_N], cntp[0, :_N, :1], cntp[1, :_N, :1], r2)

    return out8[:, :2]
